# Initial kernel scaffold; baseline (speedup 1.0000x reference)
import jax
import jax.numpy as jnp
from jax.experimental import pallas as pl


def kernel(x, edge_index, W_up, W_lw):
    raise NotImplementedError("write your pallas kernel here")


# speedup vs baseline: 13.6361x; 13.6361x over previous
import functools

import jax
import jax.numpy as jnp
from jax import lax
from jax.experimental import pallas as pl
from jax.experimental.pallas import tpu as pltpu
from jax.experimental.pallas import tpu_sc as plsc

N = 10000
E = 320000
F = 128
NPAD = 10240
NC, NS = 2, 16
NW = NC * NS
K = 128
EPAD = 327680
NB = 16
RQ = NPAD // NS
MIN_NORM = 1e-15
PROJ_EPS = 4e-3

_mesh = functools.partial(
    plsc.VectorSubcoreMesh,
    core_axis_name="c", subcore_axis_name="s", num_cores=NC, num_subcores=NS)


def _zero_rows(rows_v, nrows, width):
    def body(i, carry):
        for j in range(width // 16):
            rows_v[i, pl.ds(j * 16, 16)] = jnp.zeros((16,), jnp.float32)
        return carry
    lax.fori_loop(0, nrows, body, 0)


def _sc_counts(ei_r):
    @functools.partial(
        pl.kernel,
        out_type=jax.ShapeDtypeStruct((NC, 2, NPAD), jnp.float32),
        mesh=_mesh(),
        scratch_types=[
            pltpu.VMEM((NB, K), jnp.int32),
            pltpu.VMEM((NB, K), jnp.int32),
            pltpu.VMEM((K,), jnp.float32),
            pltpu.VMEM((K,), jnp.float32),
            pltpu.VMEM((RQ,), jnp.float32),
            pltpu.VMEM_SHARED((NPAD,), jnp.float32),
            pltpu.VMEM_SHARED((NPAD,), jnp.float32),
        ],
    )
    def kern(ei_hbm, out_hbm, srcb_v, dstb_v, ones_v, eq_v, zero_v,
             accd_sh, accl_sh):
        c = lax.axis_index("c")
        s = lax.axis_index("s")
        w = c * NS + s
        for i in range(RQ // 16):
            zero_v[pl.ds(i * 16, 16)] = jnp.zeros((16,), jnp.float32)
        for i in range(K // 16):
            ones_v[pl.ds(i * 16, 16)] = jnp.ones((16,), jnp.float32)
        pltpu.sync_copy(zero_v, accd_sh.at[pl.ds(s * RQ, RQ)])
        pltpu.sync_copy(zero_v, accl_sh.at[pl.ds(s * RQ, RQ)])
        plsc.subcore_barrier()
        nch = EPAD // K // NW
        def sb_body(sb, carry):
            lo = w * nch + sb * NB
            pltpu.sync_copy(ei_hbm.at[0, pl.ds(lo, NB), :], srcb_v)
            pltpu.sync_copy(ei_hbm.at[1, pl.ds(lo, NB), :], dstb_v)
            def ch_body(j, carry2):
                for i in range(K // 16):
                    sv = srcb_v[j, pl.ds(i * 16, 16)]
                    dv = dstb_v[j, pl.ds(i * 16, 16)]
                    eq_v[pl.ds(i * 16, 16)] = jnp.where(
                        sv == dv, 1.0, 0.0).astype(jnp.float32)
                pltpu.sync_copy(ones_v, accd_sh.at[srcb_v.at[j]], add=True)
                pltpu.sync_copy(eq_v, accl_sh.at[srcb_v.at[j]], add=True)
                return carry2
            return lax.fori_loop(0, NB, ch_body, carry)
        lax.fori_loop(0, nch // NB, sb_body, 0)
        plsc.subcore_barrier()
        pltpu.sync_copy(accd_sh.at[pl.ds(s * RQ, RQ)],
                        out_hbm.at[c, 0, pl.ds(s * RQ, RQ)])
        pltpu.sync_copy(accl_sh.at[pl.ds(s * RQ, RQ)],
                        out_hbm.at[c, 1, pl.ds(s * RQ, RQ)])
    return kern(ei_r)


def _sc_scatter(tab, ei_r, full_e):
    @functools.partial(
        pl.kernel,
        out_type=jax.ShapeDtypeStruct((NC, NPAD, F), jnp.float32),
        mesh=_mesh(),
        scratch_types=[
            pltpu.VMEM((NB, K), jnp.int32),
            pltpu.VMEM((NB, K), jnp.int32),
            pltpu.VMEM((K, F), jnp.float32),
            pltpu.VMEM_SHARED((NPAD, F), jnp.float32),
            pltpu.SemaphoreType.DMA,
        ],
    )
    def kern(tab_hbm, ei_hbm, out_hbm, srcb_v, dstb_v, rows_v, acc_sh, sem):
        c = lax.axis_index("c")
        s = lax.axis_index("s")
        _zero_rows(rows_v, K, F)
        for b in range(RQ // K):
            pltpu.sync_copy(rows_v, acc_sh.at[pl.ds(s * RQ + b * K, K)])
        plsc.subcore_barrier()
        nch = (EPAD // K // NS) if full_e else (EPAD // K // NW)
        w = s if full_e else c * NS + s
        def sb_body(sb, carry):
            lo = w * nch + sb * NB
            pltpu.sync_copy(ei_hbm.at[0, pl.ds(lo, NB), :], srcb_v)
            pltpu.sync_copy(ei_hbm.at[1, pl.ds(lo, NB), :], dstb_v)
            if full_e:
                off = c * NPAD
                def add_off(j, carry2):
                    for i in range(K // 16):
                        srcb_v[j, pl.ds(i * 16, 16)] = (
                            srcb_v[j, pl.ds(i * 16, 16)] + off)
                    return carry2
                lax.fori_loop(0, NB, add_off, 0)
            def ch_body(j, carry2):
                pltpu.async_copy(
                    tab_hbm.at[srcb_v.at[j]], rows_v, sem).wait()
                pltpu.sync_copy(rows_v, acc_sh.at[dstb_v.at[j]], add=True)
                return carry2
            return lax.fori_loop(0, NB, ch_body, carry)
        lax.fori_loop(0, nch // NB, sb_body, 0)
        plsc.subcore_barrier()
        pltpu.sync_copy(acc_sh.at[pl.ds(s * RQ, RQ)],
                        out_hbm.at[c, pl.ds(s * RQ, RQ)])
    return kern(tab, ei_r)


_R = 512
_GRID = NPAD // _R


def _tc_a(xp, w_up, cnt):
    def body(x_ref, w_ref, cnt_ref, yux_ref, xtan_ref, aux_ref):
        x = x_ref[...]
        pn = jnp.maximum(jnp.sqrt(jnp.sum(x * x, -1, keepdims=True)),
                         MIN_NORM)
        z = jnp.clip(pn, -1.0 + 1e-7, 1.0 - 1e-7)
        at = 0.5 * jnp.log((1.0 + z) / (1.0 - z))
        xt = x / pn * at
        up = lax.dot_general(xt, w_ref[...], (((1,), (1,)), ((), ())),
                             preferred_element_type=jnp.float32)
        ux = jnp.where(up >= 0, up, 0.01 * up)
        cnt = cnt_ref[...]
        deg = cnt[0, 0] + cnt[1, 0]
        nl = cnt[0, 1] + cnt[1, 1]
        dinv = jnp.where(deg > 0, lax.rsqrt(deg), 0.0)
        coef = 1.0 - dinv * dinv * jnp.where(nl > 0, 1.0, 0.0)
        yux_ref[0, ...] = dinv[:, None] * xt
        yux_ref[1, ...] = ux
        xtan_ref[...] = xt
        aux_ref[...] = jnp.stack([dinv, coef, nl])
    return pl.pallas_call(
        body,
        grid=(_GRID,),
        in_specs=[
            pl.BlockSpec((_R, F), lambda i: (i, 0)),
            pl.BlockSpec((F, F), lambda i: (0, 0)),
            pl.BlockSpec((NC, 2, _R), lambda i: (0, 0, i)),
        ],
        out_specs=[
            pl.BlockSpec((2, _R, F), lambda i: (0, i, 0)),
            pl.BlockSpec((_R, F), lambda i: (i, 0)),
            pl.BlockSpec((3, _R), lambda i: (0, i)),
        ],
        out_shape=[
            jax.ShapeDtypeStruct((2, NPAD, F), jnp.float32),
            jax.ShapeDtypeStruct((NPAD, F), jnp.float32),
            jax.ShapeDtypeStruct((3, NPAD), jnp.float32),
        ],
    )(xp, w_up, cnt)


def _tc_b(s1, yux, xtan, aux):
    def body(s1_ref, yux_ref, xtan_ref, aux_ref, usel_ref):
        s_raw = s1_ref[0]
        aux_v = aux_ref[...]
        dinv = aux_v[0][:, None]
        coef = aux_v[1][:, None]
        nl = aux_v[2][:, None]
        s = s_raw - nl * yux_ref[0]
        info = coef * xtan_ref[...] - dinv * s
        score = jnp.sum(jnp.abs(info), axis=1, keepdims=True)
        usel_ref[...] = jnp.where(score > 1.0, yux_ref[1], 0.0)
    return pl.pallas_call(
        body,
        grid=(_GRID,),
        in_specs=[
            pl.BlockSpec((NC, _R, F), lambda i: (0, i, 0)),
            pl.BlockSpec((2, _R, F), lambda i: (0, i, 0)),
            pl.BlockSpec((_R, F), lambda i: (i, 0)),
            pl.BlockSpec((3, _R), lambda i: (0, i)),
        ],
        out_specs=pl.BlockSpec((_R, F), lambda i: (i, 0)),
        out_shape=jax.ShapeDtypeStruct((NPAD, F), jnp.float32),
    )(s1, yux, xtan, aux)


def _tc_c(ssel, s1, usel, w_lw):
    def body(ss_ref, s1_ref, usel_ref, wl_ref, uw_ref):
        ss = ss_ref[...]
        sum_sel = ss[0] + ss[1]
        sneigh = s1_ref[0]
        wl = wl_ref[...]
        wa = wl[0:1, :F]
        wb = wl[0:1, F:]
        wsum = (jnp.sum(sum_sel * wa, -1, keepdims=True)
                + jnp.sum(sneigh * wb, -1, keepdims=True))
        wsel = 1.0 / (1.0 + jnp.exp(-wsum))
        uw_ref[...] = wsel * usel_ref[...]
    return pl.pallas_call(
        body,
        grid=(_GRID,),
        in_specs=[
            pl.BlockSpec((NC, _R, F), lambda i: (0, i, 0)),
            pl.BlockSpec((1, _R, F), lambda i: (1, i, 0)),
            pl.BlockSpec((_R, F), lambda i: (i, 0)),
            pl.BlockSpec((1, 2 * F), lambda i: (0, 0)),
        ],
        out_specs=pl.BlockSpec((_R, F), lambda i: (i, 0)),
        out_shape=jax.ShapeDtypeStruct((NPAD, F), jnp.float32),
    )(ssel, s1, usel, w_lw)


def _tc_d(ax, yux):
    def body(ax_ref, yux_ref, out_ref):
        ax_v = ax_ref[...]
        a = jnp.maximum(ax_v[0] + ax_v[1], 0.0)
        out = yux_ref[0] + a
        un = jnp.maximum(jnp.sqrt(jnp.sum(out * out, -1, keepdims=True)),
                         MIN_NORM)
        e = jnp.tanh(un) * out / un
        n2 = jnp.maximum(jnp.sqrt(jnp.sum(e * e, -1, keepdims=True)),
                         MIN_NORM)
        maxnorm = 1.0 - PROJ_EPS
        out_ref[...] = jnp.where(n2 > maxnorm, e / n2 * maxnorm, e)
    return pl.pallas_call(
        body,
        grid=(_GRID,),
        in_specs=[
            pl.BlockSpec((NC, _R, F), lambda i: (0, i, 0)),
            pl.BlockSpec((1, _R, F), lambda i: (1, i, 0)),
        ],
        out_specs=pl.BlockSpec((_R, F), lambda i: (i, 0)),
        out_shape=jax.ShapeDtypeStruct((NPAD, F), jnp.float32),
    )(ax, yux)


def kernel(x, edge_index, W_up, W_lw):
    x = x.astype(jnp.float32)
    xp = jnp.pad(x, ((0, NPAD - N), (0, 0)))
    pad_idx = (N + jnp.arange(EPAD - E, dtype=jnp.int32) % (NPAD - N))
    ei_p = jnp.concatenate(
        [edge_index.astype(jnp.int32),
         jnp.broadcast_to(pad_idx, (2, EPAD - E))], axis=1)
    ei_r = ei_p.reshape(2, EPAD // K, K)
    cnt = _sc_counts(ei_r)
    yux, xtan, aux = _tc_a(xp, W_up.astype(jnp.float32), cnt)
    s1 = _sc_scatter(yux.reshape(2 * NPAD, F), ei_r, full_e=True)
    usel = _tc_b(s1, yux, xtan, aux)
    ssel = _sc_scatter(usel, ei_r, full_e=False)
    uw = _tc_c(ssel, s1, usel, W_lw.astype(jnp.float32))
    ax = _sc_scatter(uw, ei_r, full_e=False)
    outp = _tc_d(ax, yux)
    return outp[:N]


# speedup vs baseline: 19.9706x; 1.4645x over previous
import functools

import jax
import jax.numpy as jnp
from jax import lax
from jax.experimental import pallas as pl
from jax.experimental.pallas import tpu as pltpu
from jax.experimental.pallas import tpu_sc as plsc

N = 10000
E = 320000
F = 128
NPAD = 10240
NC, NS = 2, 16
NW = NC * NS
K = 128
EPAD = 327680
NB = 16
RQ = NPAD // NS
MIN_NORM = 1e-15
PROJ_EPS = 4e-3

_mesh = functools.partial(
    plsc.VectorSubcoreMesh,
    core_axis_name="c", subcore_axis_name="s", num_cores=NC, num_subcores=NS)


def _zero_rows(rows_v, nrows, width):
    def body(i, carry):
        for j in range(width // 16):
            rows_v[i, pl.ds(j * 16, 16)] = jnp.zeros((16,), jnp.float32)
        return carry
    lax.fori_loop(0, nrows, body, 0)


def _sc_counts(ei_r):
    @functools.partial(
        pl.kernel,
        out_type=jax.ShapeDtypeStruct((NC, 2, NPAD), jnp.float32),
        mesh=_mesh(),
        scratch_types=[
            pltpu.VMEM((NB, K), jnp.int32),
            pltpu.VMEM((NB, K), jnp.int32),
            pltpu.VMEM((K,), jnp.float32),
            pltpu.VMEM((NB, K), jnp.float32),
            pltpu.VMEM((RQ,), jnp.float32),
            pltpu.VMEM_SHARED((NPAD,), jnp.float32),
            pltpu.VMEM_SHARED((NPAD,), jnp.float32),
            pltpu.SemaphoreType.DMA,
        ],
    )
    def kern(ei_hbm, out_hbm, srcb_v, dstb_v, ones_v, eq_v, zero_v,
             accd_sh, accl_sh, sem):
        c = lax.axis_index("c")
        s = lax.axis_index("s")
        w = c * NS + s
        for i in range(RQ // 16):
            zero_v[pl.ds(i * 16, 16)] = jnp.zeros((16,), jnp.float32)
        for i in range(K // 16):
            ones_v[pl.ds(i * 16, 16)] = jnp.ones((16,), jnp.float32)
        pltpu.sync_copy(zero_v, accd_sh.at[pl.ds(s * RQ, RQ)])
        pltpu.sync_copy(zero_v, accl_sh.at[pl.ds(s * RQ, RQ)])
        plsc.subcore_barrier()
        nch = EPAD // K // NW
        def sb_body(sb, carry):
            lo = w * nch + sb * NB
            pltpu.sync_copy(ei_hbm.at[0, pl.ds(lo, NB), :], srcb_v)
            pltpu.sync_copy(ei_hbm.at[1, pl.ds(lo, NB), :], dstb_v)
            descs = []
            for j in range(NB):
                for i in range(K // 16):
                    sv = srcb_v[j, pl.ds(i * 16, 16)]
                    dv = dstb_v[j, pl.ds(i * 16, 16)]
                    eq_v[j, pl.ds(i * 16, 16)] = jnp.where(
                        sv == dv, 1.0, 0.0).astype(jnp.float32)
            for j in range(NB):
                descs.append(pltpu.async_copy(
                    ones_v, accd_sh.at[srcb_v.at[j]], sem, add=True))
                descs.append(pltpu.async_copy(
                    eq_v.at[j], accl_sh.at[srcb_v.at[j]], sem, add=True))
            for d in descs:
                d.wait()
            return carry
        lax.fori_loop(0, nch // NB, sb_body, 0)
        plsc.subcore_barrier()
        pltpu.sync_copy(accd_sh.at[pl.ds(s * RQ, RQ)],
                        out_hbm.at[c, 0, pl.ds(s * RQ, RQ)])
        pltpu.sync_copy(accl_sh.at[pl.ds(s * RQ, RQ)],
                        out_hbm.at[c, 1, pl.ds(s * RQ, RQ)])
    return kern(ei_r)


def _sc_scatter(tab, ei_r, full_e):
    @functools.partial(
        pl.kernel,
        out_type=jax.ShapeDtypeStruct((NC, NPAD, F), jnp.float32),
        mesh=_mesh(),
        scratch_types=[
            pltpu.VMEM((NB, K), jnp.int32),
            pltpu.VMEM((NB, K), jnp.int32),
            pltpu.VMEM((2, K, F), jnp.float32),
            pltpu.VMEM_SHARED((NPAD, F), jnp.float32),
            pltpu.SemaphoreType.DMA,
            pltpu.SemaphoreType.DMA,
            pltpu.SemaphoreType.DMA,
            pltpu.SemaphoreType.DMA,
        ],
    )
    def kern(tab_hbm, ei_hbm, out_hbm, srcb_v, dstb_v, rows_v, acc_sh,
             gsem0, gsem1, ssem0, ssem1):
        c = lax.axis_index("c")
        s = lax.axis_index("s")
        gsem = (gsem0, gsem1)
        ssem = (ssem0, ssem1)
        _zero_rows(rows_v.at[0], K, F)
        for b in range(RQ // K):
            pltpu.sync_copy(rows_v.at[0],
                            acc_sh.at[pl.ds(s * RQ + b * K, K)])
        plsc.subcore_barrier()
        nch = (EPAD // K // NS) if full_e else (EPAD // K // NW)
        w = s if full_e else c * NS + s
        def sb_body(sb, carry):
            lo = w * nch + sb * NB
            pltpu.sync_copy(ei_hbm.at[0, pl.ds(lo, NB), :], srcb_v)
            pltpu.sync_copy(ei_hbm.at[1, pl.ds(lo, NB), :], dstb_v)
            if full_e:
                off = c * NPAD
                def add_off(j, carry2):
                    for i in range(K // 16):
                        srcb_v[j, pl.ds(i * 16, 16)] = (
                            srcb_v[j, pl.ds(i * 16, 16)] + off)
                    return carry2
                lax.fori_loop(0, NB, add_off, 0)
            gd = [None] * NB
            sd = [None] * NB
            gd[0] = pltpu.async_copy(
                tab_hbm.at[srcb_v.at[0]], rows_v.at[0], gsem[0])
            for j in range(NB):
                b = j % 2
                if j + 1 < NB:
                    if j >= 1:
                        sd[j - 1].wait()
                    gd[j + 1] = pltpu.async_copy(
                        tab_hbm.at[srcb_v.at[j + 1]],
                        rows_v.at[(j + 1) % 2], gsem[(j + 1) % 2])
                gd[j].wait()
                sd[j] = pltpu.async_copy(
                    rows_v.at[b], acc_sh.at[dstb_v.at[j]], ssem[b],
                    add=True)
            sd[NB - 2].wait()
            sd[NB - 1].wait()
            return carry
        lax.fori_loop(0, nch // NB, sb_body, 0)
        plsc.subcore_barrier()
        pltpu.sync_copy(acc_sh.at[pl.ds(s * RQ, RQ)],
                        out_hbm.at[c, pl.ds(s * RQ, RQ)])
    return kern(tab, ei_r)


_R = 512
_GRID = NPAD // _R


def _tc_a(xp, w_up, cnt):
    def body(x_ref, w_ref, cnt_ref, yux_ref, xtan_ref, aux_ref):
        x = x_ref[...]
        pn = jnp.maximum(jnp.sqrt(jnp.sum(x * x, -1, keepdims=True)),
                         MIN_NORM)
        z = jnp.clip(pn, -1.0 + 1e-7, 1.0 - 1e-7)
        at = 0.5 * jnp.log((1.0 + z) / (1.0 - z))
        xt = x / pn * at
        up = lax.dot_general(xt, w_ref[...], (((1,), (1,)), ((), ())),
                             preferred_element_type=jnp.float32)
        ux = jnp.where(up >= 0, up, 0.01 * up)
        cnt = cnt_ref[...]
        deg = cnt[0, 0] + cnt[1, 0]
        nl = cnt[0, 1] + cnt[1, 1]
        dinv = jnp.where(deg > 0, lax.rsqrt(deg), 0.0)
        coef = 1.0 - dinv * dinv * jnp.where(nl > 0, 1.0, 0.0)
        yux_ref[0, ...] = dinv[:, None] * xt
        yux_ref[1, ...] = ux
        xtan_ref[...] = xt
        aux_ref[...] = jnp.stack([dinv, coef, nl])
    return pl.pallas_call(
        body,
        grid=(_GRID,),
        in_specs=[
            pl.BlockSpec((_R, F), lambda i: (i, 0)),
            pl.BlockSpec((F, F), lambda i: (0, 0)),
            pl.BlockSpec((NC, 2, _R), lambda i: (0, 0, i)),
        ],
        out_specs=[
            pl.BlockSpec((2, _R, F), lambda i: (0, i, 0)),
            pl.BlockSpec((_R, F), lambda i: (i, 0)),
            pl.BlockSpec((3, _R), lambda i: (0, i)),
        ],
        out_shape=[
            jax.ShapeDtypeStruct((2, NPAD, F), jnp.float32),
            jax.ShapeDtypeStruct((NPAD, F), jnp.float32),
            jax.ShapeDtypeStruct((3, NPAD), jnp.float32),
        ],
    )(xp, w_up, cnt)


def _tc_b(s1, yux, xtan, aux):
    def body(s1_ref, yux_ref, xtan_ref, aux_ref, usel_ref):
        s_raw = s1_ref[0]
        aux_v = aux_ref[...]
        dinv = aux_v[0][:, None]
        coef = aux_v[1][:, None]
        nl = aux_v[2][:, None]
        s = s_raw - nl * yux_ref[0]
        info = coef * xtan_ref[...] - dinv * s
        score = jnp.sum(jnp.abs(info), axis=1, keepdims=True)
        usel_ref[...] = jnp.where(score > 1.0, yux_ref[1], 0.0)
    return pl.pallas_call(
        body,
        grid=(_GRID,),
        in_specs=[
            pl.BlockSpec((1, _R, F), lambda i: (0, i, 0)),
            pl.BlockSpec((2, _R, F), lambda i: (0, i, 0)),
            pl.BlockSpec((_R, F), lambda i: (i, 0)),
            pl.BlockSpec((3, _R), lambda i: (0, i)),
        ],
        out_specs=pl.BlockSpec((_R, F), lambda i: (i, 0)),
        out_shape=jax.ShapeDtypeStruct((NPAD, F), jnp.float32),
    )(s1, yux, xtan, aux)


def _tc_c(ssel, s1, usel, w_lw):
    def body(ss_ref, s1_ref, usel_ref, wl_ref, uw_ref):
        ss = ss_ref[...]
        sum_sel = ss[0] + ss[1]
        sneigh = s1_ref[0]
        wl = wl_ref[...]
        wa = wl[0:1, :F]
        wb = wl[0:1, F:]
        wsum = (jnp.sum(sum_sel * wa, -1, keepdims=True)
                + jnp.sum(sneigh * wb, -1, keepdims=True))
        wsel = 1.0 / (1.0 + jnp.exp(-wsum))
        uw_ref[...] = wsel * usel_ref[...]
    return pl.pallas_call(
        body,
        grid=(_GRID,),
        in_specs=[
            pl.BlockSpec((NC, _R, F), lambda i: (0, i, 0)),
            pl.BlockSpec((1, _R, F), lambda i: (1, i, 0)),
            pl.BlockSpec((_R, F), lambda i: (i, 0)),
            pl.BlockSpec((1, 2 * F), lambda i: (0, 0)),
        ],
        out_specs=pl.BlockSpec((_R, F), lambda i: (i, 0)),
        out_shape=jax.ShapeDtypeStruct((NPAD, F), jnp.float32),
    )(ssel, s1, usel, w_lw)


def _tc_d(ax, yux):
    def body(ax_ref, yux_ref, out_ref):
        ax_v = ax_ref[...]
        a = jnp.maximum(ax_v[0] + ax_v[1], 0.0)
        out = yux_ref[0] + a
        un = jnp.maximum(jnp.sqrt(jnp.sum(out * out, -1, keepdims=True)),
                         MIN_NORM)
        e = jnp.tanh(un) * out / un
        n2 = jnp.maximum(jnp.sqrt(jnp.sum(e * e, -1, keepdims=True)),
                         MIN_NORM)
        maxnorm = 1.0 - PROJ_EPS
        out_ref[...] = jnp.where(n2 > maxnorm, e / n2 * maxnorm, e)
    return pl.pallas_call(
        body,
        grid=(_GRID,),
        in_specs=[
            pl.BlockSpec((NC, _R, F), lambda i: (0, i, 0)),
            pl.BlockSpec((1, _R, F), lambda i: (1, i, 0)),
        ],
        out_specs=pl.BlockSpec((_R, F), lambda i: (i, 0)),
        out_shape=jax.ShapeDtypeStruct((NPAD, F), jnp.float32),
    )(ax, yux)


def kernel(x, edge_index, W_up, W_lw):
    x = x.astype(jnp.float32)
    xp = jnp.pad(x, ((0, NPAD - N), (0, 0)))
    pad_idx = (N + jnp.arange(EPAD - E, dtype=jnp.int32) % (NPAD - N))
    ei_p = jnp.concatenate(
        [edge_index.astype(jnp.int32),
         jnp.broadcast_to(pad_idx, (2, EPAD - E))], axis=1)
    ei_r = ei_p.reshape(2, EPAD // K, K)
    cnt = _sc_counts(ei_r)
    yux, xtan, aux = _tc_a(xp, W_up.astype(jnp.float32), cnt)
    s1 = _sc_scatter(yux.reshape(2 * NPAD, F), ei_r, full_e=True)
    usel = _tc_b(s1, yux, xtan, aux)
    ssel = _sc_scatter(usel, ei_r, full_e=False)
    uw = _tc_c(ssel, s1, usel, W_lw.astype(jnp.float32))
    ax = _sc_scatter(uw, ei_r, full_e=False)
    outp = _tc_d(ax, yux)
    return outp[:N]


# speedup vs baseline: 20.6435x; 1.0337x over previous
import functools

import jax
import jax.numpy as jnp
from jax import lax
from jax.experimental import pallas as pl
from jax.experimental.pallas import tpu as pltpu
from jax.experimental.pallas import tpu_sc as plsc

N = 10000
E = 320000
F = 128
NPAD = 10240
NC, NS = 2, 16
NW = NC * NS
K = 128
EPAD = 327680
NB = 16
RQ = NPAD // NS
MIN_NORM = 1e-15
PROJ_EPS = 4e-3

_mesh = functools.partial(
    plsc.VectorSubcoreMesh,
    core_axis_name="c", subcore_axis_name="s", num_cores=NC, num_subcores=NS)


def _zero_rows(rows_v, nrows, width):
    def body(i, carry):
        for j in range(width // 16):
            rows_v[i, pl.ds(j * 16, 16)] = jnp.zeros((16,), jnp.float32)
        return carry
    lax.fori_loop(0, nrows, body, 0)


def _sc_counts(ei_r):
    @functools.partial(
        pl.kernel,
        out_type=jax.ShapeDtypeStruct((NC, 2, NPAD), jnp.float32),
        mesh=_mesh(),
        scratch_types=[
            pltpu.VMEM((NB, K), jnp.int32),
            pltpu.VMEM((NB, K), jnp.int32),
            pltpu.VMEM((K,), jnp.float32),
            pltpu.VMEM((NB, K), jnp.float32),
            pltpu.VMEM((RQ,), jnp.float32),
            pltpu.VMEM_SHARED((NPAD,), jnp.float32),
            pltpu.VMEM_SHARED((NPAD,), jnp.float32),
            pltpu.SemaphoreType.DMA,
        ],
    )
    def kern(ei_hbm, out_hbm, srcb_v, dstb_v, ones_v, eq_v, zero_v,
             accd_sh, accl_sh, sem):
        c = lax.axis_index("c")
        s = lax.axis_index("s")
        w = c * NS + s
        for i in range(RQ // 16):
            zero_v[pl.ds(i * 16, 16)] = jnp.zeros((16,), jnp.float32)
        for i in range(K // 16):
            ones_v[pl.ds(i * 16, 16)] = jnp.ones((16,), jnp.float32)
        pltpu.sync_copy(zero_v, accd_sh.at[pl.ds(s * RQ, RQ)])
        pltpu.sync_copy(zero_v, accl_sh.at[pl.ds(s * RQ, RQ)])
        plsc.subcore_barrier()
        nch = EPAD // K // NW
        def sb_body(sb, carry):
            lo = w * nch + sb * NB
            pltpu.sync_copy(ei_hbm.at[0, pl.ds(lo, NB), :], srcb_v)
            pltpu.sync_copy(ei_hbm.at[1, pl.ds(lo, NB), :], dstb_v)
            descs = []
            for j in range(NB):
                for i in range(K // 16):
                    sv = srcb_v[j, pl.ds(i * 16, 16)]
                    dv = dstb_v[j, pl.ds(i * 16, 16)]
                    eq_v[j, pl.ds(i * 16, 16)] = jnp.where(
                        sv == dv, 1.0, 0.0).astype(jnp.float32)
            for j in range(NB):
                descs.append(pltpu.async_copy(
                    ones_v, accd_sh.at[srcb_v.at[j]], sem, add=True))
                descs.append(pltpu.async_copy(
                    eq_v.at[j], accl_sh.at[srcb_v.at[j]], sem, add=True))
            for d in descs:
                d.wait()
            return carry
        lax.fori_loop(0, nch // NB, sb_body, 0)
        plsc.subcore_barrier()
        pltpu.sync_copy(accd_sh.at[pl.ds(s * RQ, RQ)],
                        out_hbm.at[c, 0, pl.ds(s * RQ, RQ)])
        pltpu.sync_copy(accl_sh.at[pl.ds(s * RQ, RQ)],
                        out_hbm.at[c, 1, pl.ds(s * RQ, RQ)])
    return kern(ei_r)


NBUF = 2


def _sc_scatter(tab, ei_r, full_e):
    @functools.partial(
        pl.kernel,
        out_type=jax.ShapeDtypeStruct((NC, NPAD, F), jnp.float32),
        mesh=_mesh(),
        scratch_types=[
            pltpu.VMEM((2, NB, K), jnp.int32),
            pltpu.VMEM((2, NB, K), jnp.int32),
            pltpu.VMEM((NBUF, K, F), jnp.float32),
            pltpu.VMEM_SHARED((NPAD, F), jnp.float32),
            [pltpu.SemaphoreType.DMA] * NBUF,
            [pltpu.SemaphoreType.DMA] * NBUF,
            pltpu.SemaphoreType.DMA,
        ],
    )
    def kern(tab_hbm, ei_hbm, out_hbm, srcb_v, dstb_v, rows_v, acc_sh,
             gsem, ssem, isem):
        c = lax.axis_index("c")
        s = lax.axis_index("s")
        _zero_rows(rows_v.at[0], K, F)
        for b in range(RQ // K):
            pltpu.sync_copy(rows_v.at[0],
                            acc_sh.at[pl.ds(s * RQ + b * K, K)])
        plsc.subcore_barrier()
        nch = (EPAD // K // NS) if full_e else (EPAD // K // NW)
        w = s if full_e else c * NS + s
        nsb = nch // NB

        def load_idx(sb, p):
            lo = w * nch + sb * NB
            pltpu.async_copy(ei_hbm.at[0, pl.ds(lo, NB), :],
                             srcb_v.at[p], isem)
            pltpu.async_copy(ei_hbm.at[1, pl.ds(lo, NB), :],
                             dstb_v.at[p], isem)

        def drain_idx(p):
            pltpu.make_async_copy(
                ei_hbm.at[0, pl.ds(0, NB), :], srcb_v.at[p], isem).wait()
            pltpu.make_async_copy(
                ei_hbm.at[1, pl.ds(0, NB), :], dstb_v.at[p], isem).wait()

        load_idx(0, 0)

        def pipeline(src_i, dst_i):
            gd = [None] * NB
            sd = [None] * NB
            for j in range(min(NBUF - 1, NB)):
                gd[j] = pltpu.async_copy(
                    tab_hbm.at[src_i.at[j]], rows_v.at[j % NBUF],
                    gsem[j % NBUF])
            for j in range(NB):
                b = j % NBUF
                if j + NBUF - 1 < NB:
                    if j >= 1:
                        sd[j - 1].wait()
                    gd[j + NBUF - 1] = pltpu.async_copy(
                        tab_hbm.at[src_i.at[j + NBUF - 1]],
                        rows_v.at[(j + NBUF - 1) % NBUF],
                        gsem[(j + NBUF - 1) % NBUF])
                gd[j].wait()
                sd[j] = pltpu.async_copy(
                    rows_v.at[b], acc_sh.at[dst_i.at[j]], ssem[b],
                    add=True)
            for j in range(max(0, NB - NBUF), NB):
                sd[j].wait()

        def sb_body(sb, carry):
            p = lax.rem(sb, 2)
            drain_idx(p)
            @pl.when(sb + 1 < nsb)
            def _():
                load_idx(sb + 1, lax.rem(sb + 1, 2))
            src_i = srcb_v.at[p]
            dst_i = dstb_v.at[p]
            if full_e:
                off = c * NPAD
                def add_off(j, carry2):
                    for i in range(K // 16):
                        src_i[j, pl.ds(i * 16, 16)] = (
                            src_i[j, pl.ds(i * 16, 16)] + off)
                    return carry2
                lax.fori_loop(0, NB, add_off, 0)
            pipeline(src_i, dst_i)
            return carry
        lax.fori_loop(0, nsb, sb_body, 0)
        plsc.subcore_barrier()
        pltpu.sync_copy(acc_sh.at[pl.ds(s * RQ, RQ)],
                        out_hbm.at[c, pl.ds(s * RQ, RQ)])
    return kern(tab, ei_r)


_R = 512
_GRID = NPAD // _R


def _tc_a1(xp, w_up):
    def body(x_ref, w_ref, xtan_ref, ux_ref):
        x = x_ref[...]
        pn = jnp.maximum(jnp.sqrt(jnp.sum(x * x, -1, keepdims=True)),
                         MIN_NORM)
        z = jnp.clip(pn, -1.0 + 1e-7, 1.0 - 1e-7)
        at = 0.5 * jnp.log((1.0 + z) / (1.0 - z))
        xt = x / pn * at
        up = lax.dot_general(xt, w_ref[...], (((1,), (1,)), ((), ())),
                             preferred_element_type=jnp.float32)
        xtan_ref[...] = xt
        ux_ref[...] = jnp.where(up >= 0, up, 0.01 * up)
    return pl.pallas_call(
        body,
        grid=(_GRID,),
        in_specs=[
            pl.BlockSpec((_R, F), lambda i: (i, 0)),
            pl.BlockSpec((F, F), lambda i: (0, 0)),
        ],
        out_specs=[
            pl.BlockSpec((_R, F), lambda i: (i, 0)),
            pl.BlockSpec((_R, F), lambda i: (i, 0)),
        ],
        out_shape=[
            jax.ShapeDtypeStruct((NPAD, F), jnp.float32),
            jax.ShapeDtypeStruct((NPAD, F), jnp.float32),
        ],
    )(xp, w_up)


def _tc_a2(xtan, ux, cnt):
    def body(xtan_ref, ux_ref, cnt_ref, yux_ref, aux_ref):
        cnt = cnt_ref[...]
        deg = cnt[0, 0] + cnt[1, 0]
        nl = cnt[0, 1] + cnt[1, 1]
        dinv = jnp.where(deg > 0, lax.rsqrt(deg), 0.0)
        coef = 1.0 - dinv * dinv * jnp.where(nl > 0, 1.0, 0.0)
        yux_ref[0, ...] = dinv[:, None] * xtan_ref[...]
        yux_ref[1, ...] = ux_ref[...]
        aux_ref[...] = jnp.stack([dinv, coef, nl])
    return pl.pallas_call(
        body,
        grid=(_GRID,),
        in_specs=[
            pl.BlockSpec((_R, F), lambda i: (i, 0)),
            pl.BlockSpec((_R, F), lambda i: (i, 0)),
            pl.BlockSpec((NC, 2, _R), lambda i: (0, 0, i)),
        ],
        out_specs=[
            pl.BlockSpec((2, _R, F), lambda i: (0, i, 0)),
            pl.BlockSpec((3, _R), lambda i: (0, i)),
        ],
        out_shape=[
            jax.ShapeDtypeStruct((2, NPAD, F), jnp.float32),
            jax.ShapeDtypeStruct((3, NPAD), jnp.float32),
        ],
    )(xtan, ux, cnt)


def _tc_b(s1, yux, xtan, aux):
    def body(s1_ref, yux_ref, xtan_ref, aux_ref, usel_ref):
        s_raw = s1_ref[0]
        aux_v = aux_ref[...]
        dinv = aux_v[0][:, None]
        coef = aux_v[1][:, None]
        nl = aux_v[2][:, None]
        s = s_raw - nl * yux_ref[0]
        info = coef * xtan_ref[...] - dinv * s
        score = jnp.sum(jnp.abs(info), axis=1, keepdims=True)
        usel_ref[...] = jnp.where(score > 1.0, yux_ref[1], 0.0)
    return pl.pallas_call(
        body,
        grid=(_GRID,),
        in_specs=[
            pl.BlockSpec((1, _R, F), lambda i: (0, i, 0)),
            pl.BlockSpec((2, _R, F), lambda i: (0, i, 0)),
            pl.BlockSpec((_R, F), lambda i: (i, 0)),
            pl.BlockSpec((3, _R), lambda i: (0, i)),
        ],
        out_specs=pl.BlockSpec((_R, F), lambda i: (i, 0)),
        out_shape=jax.ShapeDtypeStruct((NPAD, F), jnp.float32),
    )(s1, yux, xtan, aux)


def _tc_c(ssel, s1, usel, w_lw):
    def body(ss_ref, s1_ref, usel_ref, wl_ref, uw_ref):
        ss = ss_ref[...]
        sum_sel = ss[0] + ss[1]
        sneigh = s1_ref[0]
        wl = wl_ref[...]
        wa = wl[0:1, :F]
        wb = wl[0:1, F:]
        wsum = (jnp.sum(sum_sel * wa, -1, keepdims=True)
                + jnp.sum(sneigh * wb, -1, keepdims=True))
        wsel = 1.0 / (1.0 + jnp.exp(-wsum))
        uw_ref[...] = wsel * usel_ref[...]
    return pl.pallas_call(
        body,
        grid=(_GRID,),
        in_specs=[
            pl.BlockSpec((NC, _R, F), lambda i: (0, i, 0)),
            pl.BlockSpec((1, _R, F), lambda i: (1, i, 0)),
            pl.BlockSpec((_R, F), lambda i: (i, 0)),
            pl.BlockSpec((1, 2 * F), lambda i: (0, 0)),
        ],
        out_specs=pl.BlockSpec((_R, F), lambda i: (i, 0)),
        out_shape=jax.ShapeDtypeStruct((NPAD, F), jnp.float32),
    )(ssel, s1, usel, w_lw)


def _tc_d(ax, yux):
    def body(ax_ref, yux_ref, out_ref):
        ax_v = ax_ref[...]
        a = jnp.maximum(ax_v[0] + ax_v[1], 0.0)
        out = yux_ref[0] + a
        un = jnp.maximum(jnp.sqrt(jnp.sum(out * out, -1, keepdims=True)),
                         MIN_NORM)
        e = jnp.tanh(un) * out / un
        n2 = jnp.maximum(jnp.sqrt(jnp.sum(e * e, -1, keepdims=True)),
                         MIN_NORM)
        maxnorm = 1.0 - PROJ_EPS
        out_ref[...] = jnp.where(n2 > maxnorm, e / n2 * maxnorm, e)
    return pl.pallas_call(
        body,
        grid=(_GRID,),
        in_specs=[
            pl.BlockSpec((NC, _R, F), lambda i: (0, i, 0)),
            pl.BlockSpec((1, _R, F), lambda i: (1, i, 0)),
        ],
        out_specs=pl.BlockSpec((_R, F), lambda i: (i, 0)),
        out_shape=jax.ShapeDtypeStruct((NPAD, F), jnp.float32),
    )(ax, yux)


def kernel(x, edge_index, W_up, W_lw):
    x = x.astype(jnp.float32)
    xp = jnp.pad(x, ((0, NPAD - N), (0, 0)))
    pad_idx = (N + jnp.arange(EPAD - E, dtype=jnp.int32) % (NPAD - N))
    ei_p = jnp.concatenate(
        [edge_index.astype(jnp.int32),
         jnp.broadcast_to(pad_idx, (2, EPAD - E))], axis=1)
    ei_r = ei_p.reshape(2, EPAD // K, K)
    cnt = _sc_counts(ei_r)
    xtan, ux = _tc_a1(xp, W_up.astype(jnp.float32))
    yux, aux = _tc_a2(xtan, ux, cnt)
    s1 = _sc_scatter(yux.reshape(2 * NPAD, F), ei_r, full_e=True)
    usel = _tc_b(s1, yux, xtan, aux)
    ssel = _sc_scatter(usel, ei_r, full_e=False)
    uw = _tc_c(ssel, s1, usel, W_lw.astype(jnp.float32))
    ax = _sc_scatter(uw, ei_r, full_e=False)
    outp = _tc_d(ax, yux)
    return outp[:N]


# speedup vs baseline: 25.8944x; 1.2544x over previous
import functools

import jax
import jax.numpy as jnp
from jax import lax
from jax.experimental import pallas as pl
from jax.experimental.pallas import tpu as pltpu
from jax.experimental.pallas import tpu_sc as plsc

N = 10000
E = 320000
F = 128
NPAD = 10240
NC, NS = 2, 16
NW = NC * NS
K = 128
EPAD = 327680
NB = 16
RQ = NPAD // NS
MIN_NORM = 1e-15
PROJ_EPS = 4e-3

_mesh = functools.partial(
    plsc.VectorSubcoreMesh,
    core_axis_name="c", subcore_axis_name="s", num_cores=NC, num_subcores=NS)


def _zero_rows(rows_v, nrows, width):
    def body(i, carry):
        for j in range(width // 16):
            rows_v[i, pl.ds(j * 16, 16)] = jnp.zeros((16,), jnp.float32)
        return carry
    lax.fori_loop(0, nrows, body, 0)


def _sc_counts(ei_r, qb):
    @functools.partial(
        pl.kernel,
        out_type=[jax.ShapeDtypeStruct((NC, 2, NPAD), jnp.float32),
                  jax.ShapeDtypeStruct((NC, 1, NPAD), jnp.float32)],
        mesh=_mesh(),
        scratch_types=[
            pltpu.VMEM((NB, K), jnp.int32),
            pltpu.VMEM((NB, K), jnp.int32),
            pltpu.VMEM((K,), jnp.float32),
            pltpu.VMEM((NB, K), jnp.float32),
            pltpu.VMEM((NB, K), jnp.float32),
            pltpu.VMEM((RQ,), jnp.float32),
            pltpu.VMEM_SHARED((NPAD,), jnp.float32),
            pltpu.VMEM_SHARED((NPAD,), jnp.float32),
            pltpu.VMEM_SHARED((NPAD,), jnp.float32),
            pltpu.SemaphoreType.DMA,
            pltpu.SemaphoreType.DMA,
        ],
    )
    def kern(ei_hbm, qb_hbm, out_hbm, outq_hbm, srcb_v, dstb_v, ones_v,
             eq_v, qb_v, zero_v, accd_sh, accl_sh, accq_sh, gsem, ssem):
        c = lax.axis_index("c")
        s = lax.axis_index("s")
        w = c * NS + s
        for i in range(RQ // 16):
            zero_v[pl.ds(i * 16, 16)] = jnp.zeros((16,), jnp.float32)
        for i in range(K // 16):
            ones_v[pl.ds(i * 16, 16)] = jnp.ones((16,), jnp.float32)
        pltpu.sync_copy(zero_v, accd_sh.at[pl.ds(s * RQ, RQ)])
        pltpu.sync_copy(zero_v, accl_sh.at[pl.ds(s * RQ, RQ)])
        pltpu.sync_copy(zero_v, accq_sh.at[pl.ds(s * RQ, RQ)])
        plsc.subcore_barrier()
        nch = EPAD // K // NW
        def sb_body(sb, carry):
            lo = w * nch + sb * NB
            pltpu.sync_copy(ei_hbm.at[0, pl.ds(lo, NB), :], srcb_v)
            pltpu.sync_copy(ei_hbm.at[1, pl.ds(lo, NB), :], dstb_v)
            gds = []
            for j in range(NB):
                gds.append(pltpu.async_copy(
                    qb_hbm.at[srcb_v.at[j]], qb_v.at[j], gsem))
            for j in range(NB):
                for i in range(K // 16):
                    sv = srcb_v[j, pl.ds(i * 16, 16)]
                    dv = dstb_v[j, pl.ds(i * 16, 16)]
                    eq_v[j, pl.ds(i * 16, 16)] = jnp.where(
                        sv == dv, 1.0, 0.0).astype(jnp.float32)
            for d in gds:
                d.wait()
            descs = []
            for j in range(NB):
                descs.append(pltpu.async_copy(
                    ones_v, accd_sh.at[srcb_v.at[j]], ssem, add=True))
                descs.append(pltpu.async_copy(
                    eq_v.at[j], accl_sh.at[srcb_v.at[j]], ssem, add=True))
                descs.append(pltpu.async_copy(
                    qb_v.at[j], accq_sh.at[dstb_v.at[j]], ssem, add=True))
            for d in descs:
                d.wait()
            return carry
        lax.fori_loop(0, nch // NB, sb_body, 0)
        plsc.subcore_barrier()
        pltpu.sync_copy(accd_sh.at[pl.ds(s * RQ, RQ)],
                        out_hbm.at[c, 0, pl.ds(s * RQ, RQ)])
        pltpu.sync_copy(accl_sh.at[pl.ds(s * RQ, RQ)],
                        out_hbm.at[c, 1, pl.ds(s * RQ, RQ)])
        pltpu.sync_copy(accq_sh.at[pl.ds(s * RQ, RQ)],
                        outq_hbm.at[c, 0, pl.ds(s * RQ, RQ)])
    return kern(ei_r, qb)


def _sc_scalar_scatter(ei_r, qa):
    @functools.partial(
        pl.kernel,
        out_type=jax.ShapeDtypeStruct((NC, 1, NPAD), jnp.float32),
        mesh=_mesh(),
        scratch_types=[
            pltpu.VMEM((NB, K), jnp.int32),
            pltpu.VMEM((NB, K), jnp.int32),
            pltpu.VMEM((NB, K), jnp.float32),
            pltpu.VMEM((RQ,), jnp.float32),
            pltpu.VMEM_SHARED((NPAD,), jnp.float32),
            pltpu.SemaphoreType.DMA,
            pltpu.SemaphoreType.DMA,
        ],
    )
    def kern(ei_hbm, qa_hbm, outq_hbm, srcb_v, dstb_v, qa_v, zero_v,
             accq_sh, gsem, ssem):
        c = lax.axis_index("c")
        s = lax.axis_index("s")
        w = c * NS + s
        for i in range(RQ // 16):
            zero_v[pl.ds(i * 16, 16)] = jnp.zeros((16,), jnp.float32)
        pltpu.sync_copy(zero_v, accq_sh.at[pl.ds(s * RQ, RQ)])
        plsc.subcore_barrier()
        nch = EPAD // K // NW
        def sb_body(sb, carry):
            lo = w * nch + sb * NB
            pltpu.sync_copy(ei_hbm.at[0, pl.ds(lo, NB), :], srcb_v)
            pltpu.sync_copy(ei_hbm.at[1, pl.ds(lo, NB), :], dstb_v)
            gds = []
            for j in range(NB):
                gds.append(pltpu.async_copy(
                    qa_hbm.at[srcb_v.at[j]], qa_v.at[j], gsem))
            for d in gds:
                d.wait()
            descs = []
            for j in range(NB):
                descs.append(pltpu.async_copy(
                    qa_v.at[j], accq_sh.at[dstb_v.at[j]], ssem, add=True))
            for d in descs:
                d.wait()
            return carry
        lax.fori_loop(0, nch // NB, sb_body, 0)
        plsc.subcore_barrier()
        pltpu.sync_copy(accq_sh.at[pl.ds(s * RQ, RQ)],
                        outq_hbm.at[c, 0, pl.ds(s * RQ, RQ)])
    return kern(ei_r, qa)


NBUF = 2


def _sc_scatter(tab, ei_r):
    @functools.partial(
        pl.kernel,
        out_type=jax.ShapeDtypeStruct((NC, NPAD, F), jnp.float32),
        mesh=_mesh(),
        scratch_types=[
            pltpu.VMEM((2, NB, K), jnp.int32),
            pltpu.VMEM((2, NB, K), jnp.int32),
            pltpu.VMEM((NBUF, K, F), jnp.float32),
            pltpu.VMEM_SHARED((NPAD, F), jnp.float32),
            [pltpu.SemaphoreType.DMA] * NBUF,
            [pltpu.SemaphoreType.DMA] * NBUF,
            pltpu.SemaphoreType.DMA,
        ],
    )
    def kern(tab_hbm, ei_hbm, out_hbm, srcb_v, dstb_v, rows_v, acc_sh,
             gsem, ssem, isem):
        c = lax.axis_index("c")
        s = lax.axis_index("s")
        _zero_rows(rows_v.at[0], K, F)
        for b in range(RQ // K):
            pltpu.sync_copy(rows_v.at[0],
                            acc_sh.at[pl.ds(s * RQ + b * K, K)])
        plsc.subcore_barrier()
        nch = EPAD // K // NW
        w = c * NS + s
        nsb = nch // NB

        def load_idx(sb, p):
            lo = w * nch + sb * NB
            pltpu.async_copy(ei_hbm.at[0, pl.ds(lo, NB), :],
                             srcb_v.at[p], isem)
            pltpu.async_copy(ei_hbm.at[1, pl.ds(lo, NB), :],
                             dstb_v.at[p], isem)

        def drain_idx(p):
            pltpu.make_async_copy(
                ei_hbm.at[0, pl.ds(0, NB), :], srcb_v.at[p], isem).wait()
            pltpu.make_async_copy(
                ei_hbm.at[1, pl.ds(0, NB), :], dstb_v.at[p], isem).wait()

        load_idx(0, 0)

        def pipeline(src_i, dst_i):
            gd = [None] * NB
            sd = [None] * NB
            for j in range(min(NBUF - 1, NB)):
                gd[j] = pltpu.async_copy(
                    tab_hbm.at[src_i.at[j]], rows_v.at[j % NBUF],
                    gsem[j % NBUF])
            for j in range(NB):
                b = j % NBUF
                if j + NBUF - 1 < NB:
                    if j >= 1:
                        sd[j - 1].wait()
                    gd[j + NBUF - 1] = pltpu.async_copy(
                        tab_hbm.at[src_i.at[j + NBUF - 1]],
                        rows_v.at[(j + NBUF - 1) % NBUF],
                        gsem[(j + NBUF - 1) % NBUF])
                gd[j].wait()
                sd[j] = pltpu.async_copy(
                    rows_v.at[b], acc_sh.at[dst_i.at[j]], ssem[b],
                    add=True)
            for j in range(max(0, NB - NBUF), NB):
                sd[j].wait()

        def sb_body(sb, carry):
            p = lax.rem(sb, 2)
            drain_idx(p)
            @pl.when(sb + 1 < nsb)
            def _():
                load_idx(sb + 1, lax.rem(sb + 1, 2))
            pipeline(srcb_v.at[p], dstb_v.at[p])
            return carry
        lax.fori_loop(0, nsb, sb_body, 0)
        plsc.subcore_barrier()
        pltpu.sync_copy(acc_sh.at[pl.ds(s * RQ, RQ)],
                        out_hbm.at[c, pl.ds(s * RQ, RQ)])
    return kern(tab, ei_r)


_R = 512
_GRID = NPAD // _R


def _tc_a1(xp, w_up, w_lw):
    def body(x_ref, w_ref, wl_ref, xtan_ref, ux_ref, qb_ref):
        x = x_ref[...]
        pn = jnp.maximum(jnp.sqrt(jnp.sum(x * x, -1, keepdims=True)),
                         MIN_NORM)
        z = jnp.clip(pn, -1.0 + 1e-7, 1.0 - 1e-7)
        at = 0.5 * jnp.log((1.0 + z) / (1.0 - z))
        xt = x / pn * at
        up = lax.dot_general(xt, w_ref[...], (((1,), (1,)), ((), ())),
                             preferred_element_type=jnp.float32)
        ux = jnp.where(up >= 0, up, 0.01 * up)
        wb = wl_ref[...][0:1, F:]
        xtan_ref[...] = xt
        ux_ref[...] = ux
        qb_ref[...] = jnp.sum(ux * wb, -1)
    return pl.pallas_call(
        body,
        grid=(_GRID,),
        in_specs=[
            pl.BlockSpec((_R, F), lambda i: (i, 0)),
            pl.BlockSpec((F, F), lambda i: (0, 0)),
            pl.BlockSpec((1, 2 * F), lambda i: (0, 0)),
        ],
        out_specs=[
            pl.BlockSpec((_R, F), lambda i: (i, 0)),
            pl.BlockSpec((_R, F), lambda i: (i, 0)),
            pl.BlockSpec((_R,), lambda i: (i,)),
        ],
        out_shape=[
            jax.ShapeDtypeStruct((NPAD, F), jnp.float32),
            jax.ShapeDtypeStruct((NPAD, F), jnp.float32),
            jax.ShapeDtypeStruct((NPAD,), jnp.float32),
        ],
    )(xp, w_up, w_lw)


def _tc_a2(xtan, cnt):
    def body(xtan_ref, cnt_ref, y_ref, aux_ref):
        cnt = cnt_ref[...]
        deg = cnt[0, 0] + cnt[1, 0]
        nl = cnt[0, 1] + cnt[1, 1]
        dinv = jnp.where(deg > 0, lax.rsqrt(deg), 0.0)
        coef = 1.0 - dinv * dinv * jnp.where(nl > 0, 1.0, 0.0)
        y_ref[...] = dinv[:, None] * xtan_ref[...]
        aux_ref[...] = jnp.stack([dinv, coef, nl])
    return pl.pallas_call(
        body,
        grid=(_GRID,),
        in_specs=[
            pl.BlockSpec((_R, F), lambda i: (i, 0)),
            pl.BlockSpec((NC, 2, _R), lambda i: (0, 0, i)),
        ],
        out_specs=[
            pl.BlockSpec((_R, F), lambda i: (i, 0)),
            pl.BlockSpec((3, _R), lambda i: (0, i)),
        ],
        out_shape=[
            jax.ShapeDtypeStruct((NPAD, F), jnp.float32),
            jax.ShapeDtypeStruct((3, NPAD), jnp.float32),
        ],
    )(xtan, cnt)


def _tc_b(s1, y, ux, xtan, aux, w_lw):
    def body(s1_ref, y_ref, ux_ref, xtan_ref, aux_ref, wl_ref,
             usel_ref, qa_ref):
        s_raw = s1_ref[0] + s1_ref[1]
        aux_v = aux_ref[...]
        dinv = aux_v[0][:, None]
        coef = aux_v[1][:, None]
        nl = aux_v[2][:, None]
        s = s_raw - nl * y_ref[...]
        info = coef * xtan_ref[...] - dinv * s
        score = jnp.sum(jnp.abs(info), axis=1, keepdims=True)
        selm = score > 1.0
        usel = jnp.where(selm, ux_ref[...], 0.0)
        wa = wl_ref[...][0:1, :F]
        usel_ref[...] = usel
        qa_ref[...] = jnp.sum(usel * wa, -1)
    return pl.pallas_call(
        body,
        grid=(_GRID,),
        in_specs=[
            pl.BlockSpec((NC, _R, F), lambda i: (0, i, 0)),
            pl.BlockSpec((_R, F), lambda i: (i, 0)),
            pl.BlockSpec((_R, F), lambda i: (i, 0)),
            pl.BlockSpec((_R, F), lambda i: (i, 0)),
            pl.BlockSpec((3, _R), lambda i: (0, i)),
            pl.BlockSpec((1, 2 * F), lambda i: (0, 0)),
        ],
        out_specs=[
            pl.BlockSpec((_R, F), lambda i: (i, 0)),
            pl.BlockSpec((_R,), lambda i: (i,)),
        ],
        out_shape=[
            jax.ShapeDtypeStruct((NPAD, F), jnp.float32),
            jax.ShapeDtypeStruct((NPAD,), jnp.float32),
        ],
    )(s1, y, ux, xtan, aux, w_lw)


def _tc_c(qas, qbs, usel):
    def body(qas_ref, qbs_ref, usel_ref, uw_ref):
        wsum = (qas_ref[0, 0] + qas_ref[1, 0]
                + qbs_ref[0, 0] + qbs_ref[1, 0])
        wsel = 1.0 / (1.0 + jnp.exp(-wsum))
        uw_ref[...] = wsel[:, None] * usel_ref[...]
    return pl.pallas_call(
        body,
        grid=(_GRID,),
        in_specs=[
            pl.BlockSpec((NC, 1, _R), lambda i: (0, 0, i)),
            pl.BlockSpec((NC, 1, _R), lambda i: (0, 0, i)),
            pl.BlockSpec((_R, F), lambda i: (i, 0)),
        ],
        out_specs=pl.BlockSpec((_R, F), lambda i: (i, 0)),
        out_shape=jax.ShapeDtypeStruct((NPAD, F), jnp.float32),
    )(qas, qbs, usel)


def _tc_d(ax, ux):
    def body(ax_ref, ux_ref, out_ref):
        ax_v = ax_ref[...]
        a = jnp.maximum(ax_v[0] + ax_v[1], 0.0)
        out = ux_ref[...] + a
        un = jnp.maximum(jnp.sqrt(jnp.sum(out * out, -1, keepdims=True)),
                         MIN_NORM)
        e = jnp.tanh(un) * out / un
        n2 = jnp.maximum(jnp.sqrt(jnp.sum(e * e, -1, keepdims=True)),
                         MIN_NORM)
        maxnorm = 1.0 - PROJ_EPS
        out_ref[...] = jnp.where(n2 > maxnorm, e / n2 * maxnorm, e)
    return pl.pallas_call(
        body,
        grid=(_GRID,),
        in_specs=[
            pl.BlockSpec((NC, _R, F), lambda i: (0, i, 0)),
            pl.BlockSpec((_R, F), lambda i: (i, 0)),
        ],
        out_specs=pl.BlockSpec((_R, F), lambda i: (i, 0)),
        out_shape=jax.ShapeDtypeStruct((NPAD, F), jnp.float32),
    )(ax, ux)


def kernel(x, edge_index, W_up, W_lw):
    x = x.astype(jnp.float32)
    w_lw = W_lw.astype(jnp.float32)
    xp = jnp.pad(x, ((0, NPAD - N), (0, 0)))
    pad_idx = (N + jnp.arange(EPAD - E, dtype=jnp.int32) % (NPAD - N))
    ei_p = jnp.concatenate(
        [edge_index.astype(jnp.int32),
         jnp.broadcast_to(pad_idx, (2, EPAD - E))], axis=1)
    ei_r = ei_p.reshape(2, EPAD // K, K)
    xtan, ux, qb = _tc_a1(xp, W_up.astype(jnp.float32), w_lw)
    cnt, qbs = _sc_counts(ei_r, qb)
    y, aux = _tc_a2(xtan, cnt)
    s1 = _sc_scatter(y, ei_r)
    usel, qa = _tc_b(s1, y, ux, xtan, aux, w_lw)
    qas = _sc_scalar_scatter(ei_r, qa)
    uw = _tc_c(qas, qbs, usel)
    ax = _sc_scatter(uw, ei_r)
    outp = _tc_d(ax, ux)
    return outp[:N]


# speedup vs baseline: 26.7025x; 1.0312x over previous
import functools

import jax
import jax.numpy as jnp
from jax import lax
from jax.experimental import pallas as pl
from jax.experimental.pallas import tpu as pltpu
from jax.experimental.pallas import tpu_sc as plsc

N = 10000
E = 320000
F = 128
NPAD = 10240
NC, NS = 2, 16
NW = NC * NS
K = 128
EPAD = 327680
NB = 16
NB_S = 40
RQ = NPAD // NS
MIN_NORM = 1e-15
PROJ_EPS = 4e-3

_mesh = functools.partial(
    plsc.VectorSubcoreMesh,
    core_axis_name="c", subcore_axis_name="s", num_cores=NC, num_subcores=NS)


def _zero_rows(rows_v, nrows, width):
    def body(i, carry):
        for j in range(width // 16):
            rows_v[i, pl.ds(j * 16, 16)] = jnp.zeros((16,), jnp.float32)
        return carry
    lax.fori_loop(0, nrows, body, 0)


def _sc_counts(ei_r, qb):
    @functools.partial(
        pl.kernel,
        out_type=[jax.ShapeDtypeStruct((NC, 2, NPAD), jnp.float32),
                  jax.ShapeDtypeStruct((NC, 1, NPAD), jnp.float32)],
        mesh=_mesh(),
        scratch_types=[
            pltpu.VMEM((NB_S, K), jnp.int32),
            pltpu.VMEM((NB_S, K), jnp.int32),
            pltpu.VMEM((K,), jnp.float32),
            pltpu.VMEM((NB_S, K), jnp.float32),
            pltpu.VMEM((NB_S, K), jnp.float32),
            pltpu.VMEM((RQ,), jnp.float32),
            pltpu.VMEM_SHARED((NPAD,), jnp.float32),
            pltpu.VMEM_SHARED((NPAD,), jnp.float32),
            pltpu.VMEM_SHARED((NPAD,), jnp.float32),
            pltpu.SemaphoreType.DMA,
            pltpu.SemaphoreType.DMA,
        ],
    )
    def kern(ei_hbm, qb_hbm, out_hbm, outq_hbm, srcb_v, dstb_v, ones_v,
             eq_v, qb_v, zero_v, accd_sh, accl_sh, accq_sh, gsem, ssem):
        c = lax.axis_index("c")
        s = lax.axis_index("s")
        w = c * NS + s
        for i in range(RQ // 16):
            zero_v[pl.ds(i * 16, 16)] = jnp.zeros((16,), jnp.float32)
        for i in range(K // 16):
            ones_v[pl.ds(i * 16, 16)] = jnp.ones((16,), jnp.float32)
        pltpu.sync_copy(zero_v, accd_sh.at[pl.ds(s * RQ, RQ)])
        pltpu.sync_copy(zero_v, accl_sh.at[pl.ds(s * RQ, RQ)])
        pltpu.sync_copy(zero_v, accq_sh.at[pl.ds(s * RQ, RQ)])
        plsc.subcore_barrier()
        nch = EPAD // K // NW
        def sb_body(sb, carry):
            lo = w * nch + sb * NB_S
            pltpu.sync_copy(ei_hbm.at[0, pl.ds(lo, NB_S), :], srcb_v)
            pltpu.sync_copy(ei_hbm.at[1, pl.ds(lo, NB_S), :], dstb_v)
            gds = []
            for j in range(NB_S):
                gds.append(pltpu.async_copy(
                    qb_hbm.at[srcb_v.at[j]], qb_v.at[j], gsem))
            for j in range(NB_S):
                for i in range(K // 16):
                    sv = srcb_v[j, pl.ds(i * 16, 16)]
                    dv = dstb_v[j, pl.ds(i * 16, 16)]
                    eq_v[j, pl.ds(i * 16, 16)] = jnp.where(
                        sv == dv, 1.0, 0.0).astype(jnp.float32)
            for d in gds:
                d.wait()
            descs = []
            for j in range(NB_S):
                descs.append(pltpu.async_copy(
                    ones_v, accd_sh.at[srcb_v.at[j]], ssem, add=True))
                descs.append(pltpu.async_copy(
                    eq_v.at[j], accl_sh.at[srcb_v.at[j]], ssem, add=True))
                descs.append(pltpu.async_copy(
                    qb_v.at[j], accq_sh.at[dstb_v.at[j]], ssem, add=True))
            for d in descs:
                d.wait()
            return carry
        lax.fori_loop(0, nch // NB_S, sb_body, 0)
        plsc.subcore_barrier()
        pltpu.sync_copy(accd_sh.at[pl.ds(s * RQ, RQ)],
                        out_hbm.at[c, 0, pl.ds(s * RQ, RQ)])
        pltpu.sync_copy(accl_sh.at[pl.ds(s * RQ, RQ)],
                        out_hbm.at[c, 1, pl.ds(s * RQ, RQ)])
        pltpu.sync_copy(accq_sh.at[pl.ds(s * RQ, RQ)],
                        outq_hbm.at[c, 0, pl.ds(s * RQ, RQ)])
    return kern(ei_r, qb)


def _sc_scalar_scatter(ei_r, qa):
    @functools.partial(
        pl.kernel,
        out_type=jax.ShapeDtypeStruct((NC, 1, NPAD), jnp.float32),
        mesh=_mesh(),
        scratch_types=[
            pltpu.VMEM((NB_S, K), jnp.int32),
            pltpu.VMEM((NB_S, K), jnp.int32),
            pltpu.VMEM((NB_S, K), jnp.float32),
            pltpu.VMEM((RQ,), jnp.float32),
            pltpu.VMEM_SHARED((NPAD,), jnp.float32),
            pltpu.SemaphoreType.DMA,
            pltpu.SemaphoreType.DMA,
        ],
    )
    def kern(ei_hbm, qa_hbm, outq_hbm, srcb_v, dstb_v, qa_v, zero_v,
             accq_sh, gsem, ssem):
        c = lax.axis_index("c")
        s = lax.axis_index("s")
        w = c * NS + s
        for i in range(RQ // 16):
            zero_v[pl.ds(i * 16, 16)] = jnp.zeros((16,), jnp.float32)
        pltpu.sync_copy(zero_v, accq_sh.at[pl.ds(s * RQ, RQ)])
        plsc.subcore_barrier()
        nch = EPAD // K // NW
        def sb_body(sb, carry):
            lo = w * nch + sb * NB_S
            pltpu.sync_copy(ei_hbm.at[0, pl.ds(lo, NB_S), :], srcb_v)
            pltpu.sync_copy(ei_hbm.at[1, pl.ds(lo, NB_S), :], dstb_v)
            gds = []
            for j in range(NB_S):
                gds.append(pltpu.async_copy(
                    qa_hbm.at[srcb_v.at[j]], qa_v.at[j], gsem))
            for d in gds:
                d.wait()
            descs = []
            for j in range(NB_S):
                descs.append(pltpu.async_copy(
                    qa_v.at[j], accq_sh.at[dstb_v.at[j]], ssem, add=True))
            for d in descs:
                d.wait()
            return carry
        lax.fori_loop(0, nch // NB_S, sb_body, 0)
        plsc.subcore_barrier()
        pltpu.sync_copy(accq_sh.at[pl.ds(s * RQ, RQ)],
                        outq_hbm.at[c, 0, pl.ds(s * RQ, RQ)])
    return kern(ei_r, qa)


NBUF = 2


def _sc_scatter(tab, ei_r):
    @functools.partial(
        pl.kernel,
        out_type=jax.ShapeDtypeStruct((NC, NPAD, F), jnp.float32),
        mesh=_mesh(),
        scratch_types=[
            pltpu.VMEM((2, NB, K), jnp.int32),
            pltpu.VMEM((2, NB, K), jnp.int32),
            pltpu.VMEM((NBUF, K, F), jnp.float32),
            pltpu.VMEM_SHARED((NPAD, F), jnp.float32),
            [pltpu.SemaphoreType.DMA] * NBUF,
            [pltpu.SemaphoreType.DMA] * NBUF,
            pltpu.SemaphoreType.DMA,
        ],
    )
    def kern(tab_hbm, ei_hbm, out_hbm, srcb_v, dstb_v, rows_v, acc_sh,
             gsem, ssem, isem):
        c = lax.axis_index("c")
        s = lax.axis_index("s")
        _zero_rows(rows_v.at[0], K, F)
        for b in range(RQ // K):
            pltpu.sync_copy(rows_v.at[0],
                            acc_sh.at[pl.ds(s * RQ + b * K, K)])
        plsc.subcore_barrier()
        nch = EPAD // K // NW
        w = c * NS + s
        nsb = nch // NB

        def load_idx(sb, p):
            lo = w * nch + sb * NB
            pltpu.async_copy(ei_hbm.at[0, pl.ds(lo, NB), :],
                             srcb_v.at[p], isem)
            pltpu.async_copy(ei_hbm.at[1, pl.ds(lo, NB), :],
                             dstb_v.at[p], isem)

        def drain_idx(p):
            pltpu.make_async_copy(
                ei_hbm.at[0, pl.ds(0, NB), :], srcb_v.at[p], isem).wait()
            pltpu.make_async_copy(
                ei_hbm.at[1, pl.ds(0, NB), :], dstb_v.at[p], isem).wait()

        load_idx(0, 0)

        def pipeline(src_i, dst_i):
            gd = [None] * NB
            sd = [None] * NB
            for j in range(min(NBUF - 1, NB)):
                gd[j] = pltpu.async_copy(
                    tab_hbm.at[src_i.at[j]], rows_v.at[j % NBUF],
                    gsem[j % NBUF])
            for j in range(NB):
                b = j % NBUF
                if j + NBUF - 1 < NB:
                    if j >= 1:
                        sd[j - 1].wait()
                    gd[j + NBUF - 1] = pltpu.async_copy(
                        tab_hbm.at[src_i.at[j + NBUF - 1]],
                        rows_v.at[(j + NBUF - 1) % NBUF],
                        gsem[(j + NBUF - 1) % NBUF])
                gd[j].wait()
                sd[j] = pltpu.async_copy(
                    rows_v.at[b], acc_sh.at[dst_i.at[j]], ssem[b],
                    add=True)
            for j in range(max(0, NB - NBUF), NB):
                sd[j].wait()

        def sb_body(sb, carry):
            p = lax.rem(sb, 2)
            drain_idx(p)
            @pl.when(sb + 1 < nsb)
            def _():
                load_idx(sb + 1, lax.rem(sb + 1, 2))
            pipeline(srcb_v.at[p], dstb_v.at[p])
            return carry
        lax.fori_loop(0, nsb, sb_body, 0)
        plsc.subcore_barrier()
        pltpu.sync_copy(acc_sh.at[pl.ds(s * RQ, RQ)],
                        out_hbm.at[c, pl.ds(s * RQ, RQ)])
    return kern(tab, ei_r)


_R = 512
_GRID = NPAD // _R


def _tc_a1(xp, w_up, w_lw):
    def body(x_ref, w_ref, wl_ref, xtan_ref, ux_ref, qb_ref):
        x = x_ref[...]
        pn = jnp.maximum(jnp.sqrt(jnp.sum(x * x, -1, keepdims=True)),
                         MIN_NORM)
        z = jnp.clip(pn, -1.0 + 1e-7, 1.0 - 1e-7)
        at = 0.5 * jnp.log((1.0 + z) / (1.0 - z))
        xt = x / pn * at
        up = lax.dot_general(xt, w_ref[...], (((1,), (1,)), ((), ())),
                             preferred_element_type=jnp.float32)
        ux = jnp.where(up >= 0, up, 0.01 * up)
        wb = wl_ref[...][0:1, F:]
        xtan_ref[...] = xt
        ux_ref[...] = ux
        qb_ref[...] = jnp.sum(ux * wb, -1)
    return pl.pallas_call(
        body,
        grid=(_GRID,),
        in_specs=[
            pl.BlockSpec((_R, F), lambda i: (i, 0)),
            pl.BlockSpec((F, F), lambda i: (0, 0)),
            pl.BlockSpec((1, 2 * F), lambda i: (0, 0)),
        ],
        out_specs=[
            pl.BlockSpec((_R, F), lambda i: (i, 0)),
            pl.BlockSpec((_R, F), lambda i: (i, 0)),
            pl.BlockSpec((_R,), lambda i: (i,)),
        ],
        out_shape=[
            jax.ShapeDtypeStruct((NPAD, F), jnp.float32),
            jax.ShapeDtypeStruct((NPAD, F), jnp.float32),
            jax.ShapeDtypeStruct((NPAD,), jnp.float32),
        ],
    )(xp, w_up, w_lw)


def _tc_a2(xtan, cnt):
    def body(xtan_ref, cnt_ref, y_ref, aux_ref):
        cnt = cnt_ref[...]
        deg = cnt[0, 0] + cnt[1, 0]
        nl = cnt[0, 1] + cnt[1, 1]
        dinv = jnp.where(deg > 0, lax.rsqrt(deg), 0.0)
        coef = 1.0 - dinv * dinv * jnp.where(nl > 0, 1.0, 0.0)
        y_ref[...] = dinv[:, None] * xtan_ref[...]
        aux_ref[...] = jnp.stack([dinv, coef, nl])
    return pl.pallas_call(
        body,
        grid=(_GRID,),
        in_specs=[
            pl.BlockSpec((_R, F), lambda i: (i, 0)),
            pl.BlockSpec((NC, 2, _R), lambda i: (0, 0, i)),
        ],
        out_specs=[
            pl.BlockSpec((_R, F), lambda i: (i, 0)),
            pl.BlockSpec((3, _R), lambda i: (0, i)),
        ],
        out_shape=[
            jax.ShapeDtypeStruct((NPAD, F), jnp.float32),
            jax.ShapeDtypeStruct((3, NPAD), jnp.float32),
        ],
    )(xtan, cnt)


def _tc_b(s1, y, ux, xtan, aux, w_lw):
    def body(s1_ref, y_ref, ux_ref, xtan_ref, aux_ref, wl_ref,
             usel_ref, qa_ref):
        s_raw = s1_ref[0] + s1_ref[1]
        aux_v = aux_ref[...]
        dinv = aux_v[0][:, None]
        coef = aux_v[1][:, None]
        nl = aux_v[2][:, None]
        s = s_raw - nl * y_ref[...]
        info = coef * xtan_ref[...] - dinv * s
        score = jnp.sum(jnp.abs(info), axis=1, keepdims=True)
        selm = score > 1.0
        usel = jnp.where(selm, ux_ref[...], 0.0)
        wa = wl_ref[...][0:1, :F]
        usel_ref[...] = usel
        qa_ref[...] = jnp.sum(usel * wa, -1)
    return pl.pallas_call(
        body,
        grid=(_GRID,),
        in_specs=[
            pl.BlockSpec((NC, _R, F), lambda i: (0, i, 0)),
            pl.BlockSpec((_R, F), lambda i: (i, 0)),
            pl.BlockSpec((_R, F), lambda i: (i, 0)),
            pl.BlockSpec((_R, F), lambda i: (i, 0)),
            pl.BlockSpec((3, _R), lambda i: (0, i)),
            pl.BlockSpec((1, 2 * F), lambda i: (0, 0)),
        ],
        out_specs=[
            pl.BlockSpec((_R, F), lambda i: (i, 0)),
            pl.BlockSpec((_R,), lambda i: (i,)),
        ],
        out_shape=[
            jax.ShapeDtypeStruct((NPAD, F), jnp.float32),
            jax.ShapeDtypeStruct((NPAD,), jnp.float32),
        ],
    )(s1, y, ux, xtan, aux, w_lw)


def _tc_c(qas, qbs, usel):
    def body(qas_ref, qbs_ref, usel_ref, uw_ref):
        wsum = (qas_ref[0, 0] + qas_ref[1, 0]
                + qbs_ref[0, 0] + qbs_ref[1, 0])
        wsel = 1.0 / (1.0 + jnp.exp(-wsum))
        uw_ref[...] = wsel[:, None] * usel_ref[...]
    return pl.pallas_call(
        body,
        grid=(_GRID,),
        in_specs=[
            pl.BlockSpec((NC, 1, _R), lambda i: (0, 0, i)),
            pl.BlockSpec((NC, 1, _R), lambda i: (0, 0, i)),
            pl.BlockSpec((_R, F), lambda i: (i, 0)),
        ],
        out_specs=pl.BlockSpec((_R, F), lambda i: (i, 0)),
        out_shape=jax.ShapeDtypeStruct((NPAD, F), jnp.float32),
    )(qas, qbs, usel)


def _tc_d(ax, ux):
    def body(ax_ref, ux_ref, out_ref):
        ax_v = ax_ref[...]
        a = jnp.maximum(ax_v[0] + ax_v[1], 0.0)
        out = ux_ref[...] + a
        un = jnp.maximum(jnp.sqrt(jnp.sum(out * out, -1, keepdims=True)),
                         MIN_NORM)
        e = jnp.tanh(un) * out / un
        n2 = jnp.maximum(jnp.sqrt(jnp.sum(e * e, -1, keepdims=True)),
                         MIN_NORM)
        maxnorm = 1.0 - PROJ_EPS
        out_ref[...] = jnp.where(n2 > maxnorm, e / n2 * maxnorm, e)
    rd = 400
    return pl.pallas_call(
        body,
        grid=(N // rd,),
        in_specs=[
            pl.BlockSpec((NC, rd, F), lambda i: (0, i, 0)),
            pl.BlockSpec((rd, F), lambda i: (i, 0)),
        ],
        out_specs=pl.BlockSpec((rd, F), lambda i: (i, 0)),
        out_shape=jax.ShapeDtypeStruct((N, F), jnp.float32),
    )(ax, ux)


def kernel(x, edge_index, W_up, W_lw):
    x = x.astype(jnp.float32)
    w_lw = W_lw.astype(jnp.float32)
    xp = jnp.pad(x, ((0, NPAD - N), (0, 0)))
    pad_idx = (N + jnp.arange(EPAD - E, dtype=jnp.int32) % (NPAD - N))
    ei_p = jnp.concatenate(
        [edge_index.astype(jnp.int32),
         jnp.broadcast_to(pad_idx, (2, EPAD - E))], axis=1)
    ei_r = ei_p.reshape(2, EPAD // K, K)
    xtan, ux, qb = _tc_a1(xp, W_up.astype(jnp.float32), w_lw)
    cnt, qbs = _sc_counts(ei_r, qb)
    y, aux = _tc_a2(xtan, cnt)
    s1 = _sc_scatter(y, ei_r)
    usel, qa = _tc_b(s1, y, ux, xtan, aux, w_lw)
    qas = _sc_scalar_scatter(ei_r, qa)
    uw = _tc_c(qas, qbs, usel)
    ax = _sc_scatter(uw, ei_r)
    return _tc_d(ax, ux)
